# Initial kernel scaffold; baseline (speedup 1.0000x reference)
import jax
import jax.numpy as jnp
from jax.experimental import pallas as pl


def kernel(item_inputs, item_embedding):
    raise NotImplementedError("write your pallas kernel here")


# speedup vs baseline: 4.5703x; 4.5703x over previous
import functools

import jax
import jax.numpy as jnp
from jax import lax
from jax.experimental import pallas as pl
from jax.experimental.pallas import tpu as pltpu
from jax.experimental.pallas import tpu_sc as plsc

NUM_ITEMS = 100000
EMBED_DIM = 64
BATCH = 4096
HIST = 50
TOTAL = BATCH * HIST

NUM_WORKERS = 32
PER_WORKER = TOTAL // NUM_WORKERS
CHUNK = 128
STREAMS = PER_WORKER // CHUNK
GROUP = 5
GROUPS = STREAMS // GROUP
GROUP_ROWS = GROUP * CHUNK


def _gather_kernel(idx_hbm, table_hbm, out_hbm, idx_v, buf, gsem):
    c = lax.axis_index("c")
    s = lax.axis_index("s")
    wid = s * 2 + c
    base = wid * PER_WORKER

    pltpu.sync_copy(idx_hbm.at[wid], idx_v)

    def group_body(g, _):
        waits = []
        for j in range(GROUP):
            waits.append(
                pltpu.async_copy(
                    table_hbm.at[idx_v.at[g * GROUP + j]],
                    buf.at[pl.ds(j * CHUNK, CHUNK)],
                    gsem,
                )
            )
        for w in waits:
            w.wait()
        pltpu.sync_copy(buf, out_hbm.at[pl.ds(base + g * GROUP_ROWS, GROUP_ROWS)])
        return _

    lax.fori_loop(0, GROUPS, group_body, None)


@functools.partial(jax.jit, static_argnames=())
def kernel(item_inputs, item_embedding):
    idx = item_inputs.astype(jnp.int32).reshape(NUM_WORKERS, STREAMS, CHUNK)
    mesh = plsc.VectorSubcoreMesh(core_axis_name="c", subcore_axis_name="s")
    out = pl.kernel(
        _gather_kernel,
        out_type=jax.ShapeDtypeStruct((TOTAL, EMBED_DIM), jnp.float32),
        mesh=mesh,
        scratch_types=[
            pltpu.VMEM((STREAMS, CHUNK), jnp.int32),
            pltpu.VMEM((GROUP_ROWS, EMBED_DIM), jnp.float32),
            pltpu.SemaphoreType.DMA,
        ],
        compiler_params=pltpu.CompilerParams(use_tc_tiling_on_sc=False),
    )(idx, item_embedding)
    return out.reshape(BATCH, HIST, EMBED_DIM)


# speedup vs baseline: 4.6253x; 1.0120x over previous
import functools

import jax
import jax.numpy as jnp
from jax import lax
from jax.experimental import pallas as pl
from jax.experimental.pallas import tpu as pltpu
from jax.experimental.pallas import tpu_sc as plsc

NUM_ITEMS = 100000
EMBED_DIM = 64
BATCH = 4096
HIST = 50
TOTAL = BATCH * HIST

NUM_WORKERS = 32
PER_WORKER = TOTAL // NUM_WORKERS
CHUNK = 128
STREAMS = PER_WORKER // CHUNK
GROUP = 5
GROUPS = STREAMS // GROUP
GROUP_ROWS = GROUP * CHUNK
PAIRS = GROUPS // 2


def _gather_kernel(idx_hbm, table_hbm, out_hbm,
                   idx_v, buf0, buf1, gsem0, gsem1, wsem0, wsem1):
    c = lax.axis_index("c")
    s = lax.axis_index("s")
    wid = s * 2 + c
    base = wid * PER_WORKER

    pltpu.sync_copy(idx_hbm.at[wid], idx_v)

    def fire(g, buf, sem):
        for j in range(GROUP):
            pltpu.async_copy(
                table_hbm.at[idx_v.at[g * GROUP + j]],
                buf.at[pl.ds(j * CHUNK, CHUNK)],
                sem,
            )

    def drain_gather(buf, sem):
        for j in range(GROUP):
            pltpu.make_async_copy(
                table_hbm.at[pl.ds(0, CHUNK)],
                buf.at[pl.ds(j * CHUNK, CHUNK)],
                sem,
            ).wait()

    def wb(g, buf, sem):
        pltpu.async_copy(
            buf, out_hbm.at[pl.ds(base + g * GROUP_ROWS, GROUP_ROWS)], sem)

    def drain_wb(buf, sem):
        pltpu.make_async_copy(
            buf, out_hbm.at[pl.ds(base, GROUP_ROWS)], sem).wait()

    fire(0, buf0, gsem0)

    def pair_body(p, carry):
        g0 = 2 * p
        drain_gather(buf0, gsem0)

        @pl.when(p > 0)
        def _():
            drain_wb(buf1, wsem1)

        fire(g0 + 1, buf1, gsem1)
        wb(g0, buf0, wsem0)

        drain_gather(buf1, gsem1)
        drain_wb(buf0, wsem0)

        @pl.when(p < PAIRS - 1)
        def _():
            fire(g0 + 2, buf0, gsem0)

        wb(g0 + 1, buf1, wsem1)
        return carry

    lax.fori_loop(0, PAIRS, pair_body, None)
    drain_wb(buf1, wsem1)


@functools.partial(jax.jit, static_argnames=())
def kernel(item_inputs, item_embedding):
    idx = item_inputs.astype(jnp.int32).reshape(NUM_WORKERS, STREAMS, CHUNK)
    mesh = plsc.VectorSubcoreMesh(core_axis_name="c", subcore_axis_name="s")
    out = pl.kernel(
        _gather_kernel,
        out_type=jax.ShapeDtypeStruct((TOTAL, EMBED_DIM), jnp.float32),
        mesh=mesh,
        scratch_types=[
            pltpu.VMEM((STREAMS, CHUNK), jnp.int32),
            pltpu.VMEM((GROUP_ROWS, EMBED_DIM), jnp.float32),
            pltpu.VMEM((GROUP_ROWS, EMBED_DIM), jnp.float32),
            pltpu.SemaphoreType.DMA,
            pltpu.SemaphoreType.DMA,
            pltpu.SemaphoreType.DMA,
            pltpu.SemaphoreType.DMA,
        ],
        compiler_params=pltpu.CompilerParams(use_tc_tiling_on_sc=False),
    )(idx, item_embedding)
    return out.reshape(BATCH, HIST, EMBED_DIM)
